# baseline (speedup 1.0000x reference)
import functools

import jax
import jax.numpy as jnp
from jax import lax
from jax.experimental import pallas as pl
from jax.experimental.pallas import tpu as pltpu
from jax.experimental.pallas import tpu_sc as plsc

_EMBED = 64
_SCALE = 8.0

_info = plsc.get_sparse_core_info()
_NC = _info.num_cores
_NS = _info.num_subcores
_L = _info.num_lanes
_NW = _NC * _NS

_CHUNK = 128
_VECS_PER_ROW = _EMBED // _L


def kernel(tokens, table):
    B = tokens.shape[0] * tokens.shape[1]
    flat = tokens.reshape((B,)).astype(jnp.int32)
    b_per_w = B // _NW
    n_chunks = b_per_w // _CHUNK

    @functools.partial(
        pl.kernel,
        mesh=plsc.VectorSubcoreMesh(core_axis_name="c", subcore_axis_name="s"),
        compiler_params=pltpu.CompilerParams(use_tc_tiling_on_sc=False),
        out_type=jax.ShapeDtypeStruct((B, _EMBED), jnp.float32),
        scratch_types=[
            pltpu.VMEM((_CHUNK,), jnp.int32),
            pltpu.VMEM((_CHUNK, _EMBED), jnp.float32),
            pltpu.SemaphoreType.DMA,
        ],
    )
    def _emb(tok_hbm, table_hbm, out_hbm, idx_v, rows_v, sem):
        wid = lax.axis_index("s") * _NC + lax.axis_index("c")
        base = wid * b_per_w

        def chunk_body(c, carry):
            off = base + c * _CHUNK
            pltpu.sync_copy(tok_hbm.at[pl.ds(off, _CHUNK)], idx_v)
            pltpu.async_copy(table_hbm.at[idx_v], rows_v, sem).wait()

            def row_body(i, carry2):
                for j in range(_VECS_PER_ROW):
                    rows_v[i, pl.ds(j * _L, _L)] = (
                        rows_v[i, pl.ds(j * _L, _L)] * _SCALE
                    )
                return carry2

            lax.fori_loop(0, _CHUNK, row_body, 0)
            pltpu.sync_copy(rows_v, out_hbm.at[pl.ds(off, _CHUNK)])
            return carry

        lax.fori_loop(0, n_chunks, chunk_body, 0)

    out = _emb(flat, table)
    return out.reshape(tokens.shape + (_EMBED,))


# speedup vs baseline: 1.0035x; 1.0035x over previous
import functools

import jax
import jax.numpy as jnp
from jax import lax
from jax.experimental import pallas as pl
from jax.experimental.pallas import tpu as pltpu
from jax.experimental.pallas import tpu_sc as plsc

_EMBED = 64
_SCALE = 8.0

_info = plsc.get_sparse_core_info()
_NC = _info.num_cores
_NS = _info.num_subcores
_L = _info.num_lanes
_NW = _NC * _NS

_CHUNK = 128
_VECS_PER_ROW = _EMBED // _L
_NBUF = 4
_AHEAD = 2
_ROW_UNROLL = 8


def kernel(tokens, table):
    B = tokens.shape[0] * tokens.shape[1]
    n_chunks_total = B // _CHUNK
    n_chunks = n_chunks_total // _NW
    tok2d = tokens.reshape((n_chunks_total, _CHUNK)).astype(jnp.int32)

    @functools.partial(
        pl.kernel,
        mesh=plsc.VectorSubcoreMesh(core_axis_name="c", subcore_axis_name="s"),
        compiler_params=pltpu.CompilerParams(use_tc_tiling_on_sc=False),
        out_type=jax.ShapeDtypeStruct((B, _EMBED), jnp.float32),
        scratch_types=[
            pltpu.VMEM((n_chunks, _CHUNK), jnp.int32),
            pltpu.VMEM((_NBUF, _CHUNK, _EMBED), jnp.float32),
            pltpu.SemaphoreType.DMA((_NBUF,)),
            pltpu.SemaphoreType.DMA((_NBUF,)),
        ],
    )
    def _emb(tok_hbm, table_hbm, out_hbm, idx_v, rows_v, gsem, osem):
        wid = lax.axis_index("s") * _NC + lax.axis_index("c")
        cbase = wid * n_chunks

        pltpu.sync_copy(tok_hbm.at[pl.ds(cbase, n_chunks)], idx_v)

        def start_gather(c, b):
            pltpu.async_copy(
                table_hbm.at[idx_v.at[c]], rows_v.at[b], gsem.at[b]
            )

        for c in range(_AHEAD):
            start_gather(c, c % _NBUF)

        def chunk_body(c, carry):
            b = lax.rem(c, _NBUF)
            ca = c + _AHEAD
            ba = lax.rem(ca, _NBUF)

            @pl.when(c >= _NBUF - _AHEAD)
            def _():
                pltpu.make_async_copy(
                    rows_v.at[ba], out_hbm.at[pl.ds(0, _CHUNK)], osem.at[ba]
                ).wait()

            @pl.when(ca < n_chunks)
            def _():
                start_gather(ca, ba)

            pltpu.make_async_copy(
                table_hbm.at[idx_v.at[c]], rows_v.at[b], gsem.at[b]
            ).wait()

            def row_body(i, carry2):
                for k in range(_ROW_UNROLL):
                    for j in range(_VECS_PER_ROW):
                        r = i * _ROW_UNROLL + k
                        rows_v[b, r, pl.ds(j * _L, _L)] = (
                            rows_v[b, r, pl.ds(j * _L, _L)] * _SCALE
                        )
                return carry2

            lax.fori_loop(0, _CHUNK // _ROW_UNROLL, row_body, 0)

            pltpu.async_copy(
                rows_v.at[b], out_hbm.at[pl.ds((cbase + c) * _CHUNK, _CHUNK)],
                osem.at[b],
            )
            return carry

        lax.fori_loop(0, n_chunks, chunk_body, 0)

        for c in range(n_chunks - (_NBUF - _AHEAD), n_chunks):
            b = c % _NBUF
            pltpu.make_async_copy(
                rows_v.at[b], out_hbm.at[pl.ds(0, _CHUNK)], osem.at[b]
            ).wait()

    out = _emb(tok2d, table)
    return out.reshape(tokens.shape + (_EMBED,))
